# baseline (speedup 1.0000x reference)
import functools

import jax
import jax.numpy as jnp
from jax import lax
from jax.experimental import pallas as pl
from jax.experimental.pallas import tpu as pltpu
from jax.experimental.pallas import tpu_sc as plsc

_INFO = plsc.get_sparse_core_info()
_NC = _INFO.num_cores
_NW = _NC * _INFO.num_subcores

_CHUNK_ROWS = 96


def kernel(img_batch, canvases):
    del canvases
    b, c, h, w = img_batch.shape
    rows = b * c * h
    flat = img_batch.reshape(rows, w)
    rpw = rows // _NW
    nchunks = rpw // _CHUNK_ROWS
    mesh = plsc.VectorSubcoreMesh(core_axis_name="c", subcore_axis_name="s")

    @functools.partial(
        pl.kernel,
        out_type=jax.ShapeDtypeStruct((rows, w), jnp.float32),
        mesh=mesh,
        scratch_types=[
            pltpu.VMEM((2, _CHUNK_ROWS, w), jnp.float32),
            pltpu.SemaphoreType.DMA((2,)),
            pltpu.SemaphoreType.DMA((2,)),
        ],
    )
    def _sc_slot_copy(src_hbm, out_hbm, buf, in_sems, out_sems):
        wid = lax.axis_index("s") * _NC + lax.axis_index("c")
        base = wid * rpw

        def in_copy(k, slot):
            return pltpu.make_async_copy(
                src_hbm.at[pl.ds(base + k * _CHUNK_ROWS, _CHUNK_ROWS)],
                buf.at[slot],
                in_sems.at[slot],
            )

        def out_copy(k, slot):
            return pltpu.make_async_copy(
                buf.at[slot],
                out_hbm.at[pl.ds(base + k * _CHUNK_ROWS, _CHUNK_ROWS)],
                out_sems.at[slot],
            )

        in_copy(0, 0).start()
        for k in range(nchunks):
            slot = k % 2
            in_copy(k, slot).wait()
            nxt = (k + 1) % 2
            if k + 1 < nchunks:
                if k >= 1:
                    out_copy(k - 1, nxt).wait()
                in_copy(k + 1, nxt).start()
            out_copy(k, slot).start()
        out_copy(nchunks - 2, (nchunks - 2) % 2).wait()
        out_copy(nchunks - 1, (nchunks - 1) % 2).wait()

    return _sc_slot_copy(flat).reshape(b, c, h, w)


# speedup vs baseline: 1.0152x; 1.0152x over previous
import functools

import jax
import jax.numpy as jnp
from jax import lax
from jax.experimental import pallas as pl
from jax.experimental.pallas import tpu as pltpu
from jax.experimental.pallas import tpu_sc as plsc

_INFO = plsc.get_sparse_core_info()
_NC = _INFO.num_cores
_NW = _NC * _INFO.num_subcores

_CHUNK_ROWS = 48
_NBUF = 4


def kernel(img_batch, canvases):
    del canvases
    b, c, h, w = img_batch.shape
    rows = b * c * h
    flat = img_batch.reshape(rows, w)
    rpw = rows // _NW
    nchunks = rpw // _CHUNK_ROWS
    mesh = plsc.VectorSubcoreMesh(core_axis_name="c", subcore_axis_name="s")

    @functools.partial(
        pl.kernel,
        out_type=jax.ShapeDtypeStruct((rows, w), jnp.float32),
        mesh=mesh,
        scratch_types=[
            pltpu.VMEM((_NBUF, _CHUNK_ROWS, w), jnp.float32),
            pltpu.SemaphoreType.DMA((_NBUF,)),
            pltpu.SemaphoreType.DMA((_NBUF,)),
        ],
    )
    def _sc_slot_copy(src_hbm, out_hbm, buf, in_sems, out_sems):
        wid = lax.axis_index("s") * _NC + lax.axis_index("c")
        base = wid * rpw

        def in_copy(k):
            return pltpu.make_async_copy(
                src_hbm.at[pl.ds(base + k * _CHUNK_ROWS, _CHUNK_ROWS)],
                buf.at[k % _NBUF],
                in_sems.at[k % _NBUF],
            )

        def out_copy(k):
            return pltpu.make_async_copy(
                buf.at[k % _NBUF],
                out_hbm.at[pl.ds(base + k * _CHUNK_ROWS, _CHUNK_ROWS)],
                out_sems.at[k % _NBUF],
            )

        for k in range(_NBUF - 1):
            in_copy(k).start()
        for k in range(nchunks):
            in_copy(k).wait()
            out_copy(k).start()
            j = k + _NBUF - 1
            if j < nchunks:
                if k >= 1:
                    out_copy(k - 1).wait()
                in_copy(j).start()
        for k in range(max(0, nchunks - _NBUF), nchunks):
            out_copy(k).wait()

    return _sc_slot_copy(flat).reshape(b, c, h, w)


# speedup vs baseline: 1.0316x; 1.0162x over previous
import functools

import jax
import jax.numpy as jnp
from jax import lax
from jax.experimental import pallas as pl
from jax.experimental.pallas import tpu as pltpu
from jax.experimental.pallas import tpu_sc as plsc

_INFO = plsc.get_sparse_core_info()
_NC = _INFO.num_cores
_NW = _NC * _INFO.num_subcores

_CHUNK_ROWS = 128
_NBUF = 2


def kernel(img_batch, canvases):
    del canvases
    b, c, h, w = img_batch.shape
    rows = b * c * h
    flat = img_batch.reshape(rows, w)
    rpw = rows // _NW
    nchunks = rpw // _CHUNK_ROWS
    mesh = plsc.VectorSubcoreMesh(core_axis_name="c", subcore_axis_name="s")

    @functools.partial(
        pl.kernel,
        out_type=jax.ShapeDtypeStruct((rows, w), jnp.float32),
        mesh=mesh,
        scratch_types=[
            pltpu.VMEM_SHARED((_NBUF, 16 * _CHUNK_ROWS, w), jnp.float32),
            pltpu.SemaphoreType.DMA((_NBUF,)),
            pltpu.SemaphoreType.DMA((_NBUF,)),
        ],
    )
    def _sc_slot_copy(src_hbm, out_hbm, buf, in_sems, out_sems):
        sid = lax.axis_index("s")
        wid = sid * _NC + lax.axis_index("c")
        base = wid * rpw

        def in_copy(k):
            return pltpu.make_async_copy(
                src_hbm.at[pl.ds(base + k * _CHUNK_ROWS, _CHUNK_ROWS)],
                buf.at[k % _NBUF, pl.ds(sid * _CHUNK_ROWS, _CHUNK_ROWS)],
                in_sems.at[k % _NBUF],
            )

        def out_copy(k):
            return pltpu.make_async_copy(
                buf.at[k % _NBUF, pl.ds(sid * _CHUNK_ROWS, _CHUNK_ROWS)],
                out_hbm.at[pl.ds(base + k * _CHUNK_ROWS, _CHUNK_ROWS)],
                out_sems.at[k % _NBUF],
            )

        for k in range(_NBUF - 1):
            in_copy(k).start()
        for k in range(nchunks):
            in_copy(k).wait()
            out_copy(k).start()
            j = k + _NBUF - 1
            if j < nchunks:
                if k >= 1:
                    out_copy(k - 1).wait()
                in_copy(j).start()
        for k in range(max(0, nchunks - _NBUF), nchunks):
            out_copy(k).wait()

    return _sc_slot_copy(flat).reshape(b, c, h, w)
